# baseline (speedup 1.0000x reference)
import functools
import math

import jax
import jax.numpy as jnp
from jax import lax
from jax.experimental import pallas as pl
from jax.experimental.pallas import tpu as pltpu
from jax.experimental.pallas import tpu_sc as plsc

D_MODEL = 64
CHUNK = 128
SCALE = math.sqrt(D_MODEL)


def _make_sc_kernel(n_flat, num_cores, num_subcores):
    n_workers = num_cores * num_subcores
    per_worker = n_flat // n_workers
    n_chunks = per_worker // CHUNK

    mesh = plsc.VectorSubcoreMesh(core_axis_name="c", subcore_axis_name="s")

    @functools.partial(
        pl.kernel,
        mesh=mesh,
        out_type=jax.ShapeDtypeStruct((n_flat, D_MODEL), jnp.float32),
        compiler_params=pltpu.CompilerParams(use_tc_tiling_on_sc=False),
        scratch_types=[
            pltpu.VMEM((n_chunks, CHUNK), jnp.int32),
            pltpu.VMEM((CHUNK, D_MODEL), jnp.float32),
            pltpu.SemaphoreType.DMA,
        ],
    )
    def k(x_hbm, lut_hbm, out_hbm, idx_v, gbuf, sem):
        wid = lax.axis_index("s") * num_cores + lax.axis_index("c")
        pltpu.sync_copy(x_hbm.at[pl.ds(wid * n_chunks, n_chunks)], idx_v)

        def chunk_body(c, carry):
            pltpu.async_copy(lut_hbm.at[idx_v.at[c]], gbuf, sem).wait()

            def row_body(r, rc):
                for d in range(D_MODEL // 16):
                    sl = pl.ds(d * 16, 16)
                    gbuf[r, sl] = gbuf[r, sl] * SCALE
                return rc

            lax.fori_loop(0, CHUNK, row_body, 0)
            base = wid * per_worker + c * CHUNK
            pltpu.sync_copy(gbuf, out_hbm.at[pl.ds(base, CHUNK)])
            return carry

        lax.fori_loop(0, n_chunks, chunk_body, 0)

    return k


def kernel(x, lut):
    xf = x.reshape(-1).astype(jnp.int32)
    n_flat = xf.shape[0]
    info = plsc.get_sparse_core_info()
    x2d = xf.reshape(n_flat // CHUNK, CHUNK)
    out = _make_sc_kernel(n_flat, info.num_cores, info.num_subcores)(x2d, lut)
    return out.reshape(*x.shape, D_MODEL)
